# baseline (speedup 1.0000x reference)
import jax
import jax.numpy as jnp
from jax import lax
from jax.experimental import pallas as pl
from jax.experimental.pallas import tpu as pltpu
from jax.experimental.pallas import tpu_sc as plsc

NC = 2
NS = 16
NW = NC * NS

N = 10000
E = 320000
D = 128
K = 6

GL = 128
GPW = 80
EP = NW * GPW * GL
NP = 10240
RPT = NP // NS
ZR = 128
CHG = 16
NCH = GPW // CHG

_mesh = plsc.VectorSubcoreMesh(core_axis_name="c", subcore_axis_name="s")


def _matvec_body(x_h, w2f_h, send3_h, recv3_h, p_h,
                 sidx_v, ridx_v, wval0_v, wval1_v, rows_v, acc_sh, gsem, ssem):
    wvals = (wval0_v, wval1_v)
    c = lax.axis_index("c")
    s = lax.axis_index("s")
    wid = s * NC + c

    def _zb(i, carry):
        for b in range(D // 16):
            rows_v[0, i, pl.ds(b * 16, 16)] = jnp.zeros((16,), jnp.float32)
        return carry
    lax.fori_loop(0, ZR, _zb, 0)
    for i in range(RPT // ZR):
        pltpu.sync_copy(rows_v.at[0], acc_sh.at[pl.ds(s * RPT + i * ZR, ZR)])

    plsc.subcore_barrier()

    def _stage(c):
        slot = c % 2
        pltpu.sync_copy(send3_h.at[wid, pl.ds(c * CHG, CHG)], sidx_v.at[slot])
        pltpu.sync_copy(recv3_h.at[wid, pl.ds(c * CHG, CHG)], ridx_v.at[slot])
        pltpu.sync_copy(w2f_h.at[wid, pl.ds(c * CHG * GL, CHG * GL)],
                        wvals[slot])

    def _issue_gather(g):
        c, i = g // CHG, g % CHG
        return pltpu.async_copy(
            x_h.at[ridx_v.at[c % 2, i]], rows_v.at[g % 2], gsem)

    def _issue_scatter(g):
        c, i = g // CHG, g % CHG
        return pltpu.async_copy(
            rows_v.at[g % 2], acc_sh.at[sidx_v.at[c % 2, i]], ssem, add=True)

    def _mul_group(g):
        slot, i = (g // CHG) % 2, g % CHG
        p = g % 2

        def _mul(r, c2):
            wb = plsc.load_gather(
                wvals[slot], [jnp.zeros((16,), jnp.int32) + (i * GL + r)])
            for b in range(D // 16):
                rows_v[p, r, pl.ds(b * 16, 16)] = (
                    rows_v[p, r, pl.ds(b * 16, 16)] * wb)
            return c2
        lax.fori_loop(0, GL, _mul, 0)

    _stage(0)
    gd = _issue_gather(0)
    sd = None
    for g in range(GPW):
        gd.wait()
        if sd is not None:
            sd.wait()
        if g + 1 < GPW:
            if g % CHG == 0 and g // CHG + 1 < NCH:
                _stage(g // CHG + 1)
            gd = _issue_gather(g + 1)
        _mul_group(g)
        sd = _issue_scatter(g)
    sd.wait()

    plsc.subcore_barrier()
    for i in range(RPT // ZR):
        off = s * RPT + i * ZR
        pltpu.sync_copy(acc_sh.at[pl.ds(off, ZR)], p_h.at[c, pl.ds(off, ZR)])


_matvec = pl.kernel(
    _matvec_body,
    out_type=jax.ShapeDtypeStruct((NC, NP, D), jnp.float32),
    mesh=_mesh,
    compiler_params=pltpu.CompilerParams(needs_layout_passes=False),
    scratch_types=[
        pltpu.VMEM((2, CHG, GL), jnp.int32),
        pltpu.VMEM((2, CHG, GL), jnp.int32),
        pltpu.VMEM((CHG * GL,), jnp.float32),
        pltpu.VMEM((CHG * GL,), jnp.float32),
        pltpu.VMEM((2, GL, D), jnp.float32),
        pltpu.VMEM_SHARED((NP, D), jnp.float32),
        pltpu.SemaphoreType.DMA,
        pltpu.SemaphoreType.DMA,
    ],
)


def _deg_body(w3_h, send3_h, pdeg_h, sidx_v, wval_v, zv_v, accd_sh):
    c = lax.axis_index("c")
    s = lax.axis_index("s")
    wid = s * NC + c

    def _zb(i, carry):
        zv_v[pl.ds(i * 16, 16)] = jnp.zeros((16,), jnp.float32)
        return carry
    lax.fori_loop(0, RPT // 16, _zb, 0)
    pltpu.sync_copy(zv_v, accd_sh.at[pl.ds(s * RPT, RPT)])

    pltpu.sync_copy(send3_h.at[wid], sidx_v)
    pltpu.sync_copy(w3_h.at[wid], wval_v)
    plsc.subcore_barrier()

    def _grp(j, carry):
        pltpu.sync_copy(wval_v.at[j], accd_sh.at[sidx_v.at[j]], add=True)
        return carry
    lax.fori_loop(0, GPW, _grp, 0)

    plsc.subcore_barrier()
    pltpu.sync_copy(accd_sh.at[pl.ds(s * RPT, RPT)],
                    pdeg_h.at[c, pl.ds(s * RPT, RPT)])


_deg = pl.kernel(
    _deg_body,
    out_type=jax.ShapeDtypeStruct((NC, NP), jnp.float32),
    mesh=_mesh,
    scratch_types=[
        pltpu.VMEM((GPW, GL), jnp.int32),
        pltpu.VMEM((GPW, GL), jnp.float32),
        pltpu.VMEM((RPT,), jnp.float32),
        pltpu.VMEM_SHARED((NP,), jnp.float32),
    ],
)


BM = 1000


def _scale_body(pdeg_ref, edges_ref, sdeg_ref, scale_ref):
    deg = pdeg_ref[0, :] + pdeg_ref[1, :]
    m = jnp.maximum(jnp.max(deg), jnp.max(-edges_ref[...]))
    sc = 1.0 / m
    scale_ref[0, 0] = sc
    sdeg_ref[...] = deg * sc


def _scale_call(pdeg, edges):
    return pl.pallas_call(
        _scale_body,
        out_shape=[
            jax.ShapeDtypeStruct((NP,), jnp.float32),
            jax.ShapeDtypeStruct((1, 1), jnp.float32),
        ],
        out_specs=[
            pl.BlockSpec(memory_space=pltpu.VMEM),
            pl.BlockSpec(memory_space=pltpu.SMEM),
        ],
    )(pdeg, edges)


def _combine1_body(scale_ref, sdeg_ref, x_ref, p_ref, y_ref):
    sc = scale_ref[0, 0]
    ax = p_ref[0] + p_ref[1]
    y_ref[...] = sdeg_ref[...] * x_ref[...] - sc * ax


def _combine2_body(scale_ref, sdeg_ref, x_ref, p_ref, prev_ref, y_ref):
    sc = scale_ref[0, 0]
    ax = p_ref[0] + p_ref[1]
    y_ref[...] = 2.0 * (sdeg_ref[...] * x_ref[...] - sc * ax) - prev_ref[...]


def _combine(scale, sdeg2, x, p, prev=None):
    grid = (N // BM,)
    scale_spec = pl.BlockSpec(memory_space=pltpu.SMEM)
    sdeg_spec = pl.BlockSpec((BM, 1), lambda i: (i, 0))
    row_spec = pl.BlockSpec((BM, D), lambda i: (i, 0))
    p_spec = pl.BlockSpec((NC, BM, D), lambda i: (0, i, 0))
    if prev is None:
        return pl.pallas_call(
            _combine1_body,
            grid=grid,
            in_specs=[scale_spec, sdeg_spec, row_spec, p_spec],
            out_specs=row_spec,
            out_shape=jax.ShapeDtypeStruct((N, D), jnp.float32),
        )(scale, sdeg2, x, p)
    return pl.pallas_call(
        _combine2_body,
        grid=grid,
        in_specs=[scale_spec, sdeg_spec, row_spec, p_spec, row_spec],
        out_specs=row_spec,
        out_shape=jax.ShapeDtypeStruct((N, D), jnp.float32),
    )(scale, sdeg2, x, p, prev)


def _matmul_body(x_ref, w_ref, db_ref, b_ref, o_ref):
    acc = jnp.dot(x_ref[...], w_ref[...], preferred_element_type=jnp.float32)
    o_ref[...] = acc + jnp.sum(db_ref[...], axis=0, keepdims=True) + b_ref[...]


def _matmul(xs, wf, dense_b, bias2):
    grid = (N // BM,)
    return pl.pallas_call(
        _matmul_body,
        grid=grid,
        in_specs=[
            pl.BlockSpec((BM, K * D), lambda i: (i, 0)),
            pl.BlockSpec((K * D, D), lambda i: (0, 0)),
            pl.BlockSpec((K, D), lambda i: (0, 0)),
            pl.BlockSpec((1, D), lambda i: (0, 0)),
        ],
        out_specs=pl.BlockSpec((BM, D), lambda i: (i, 0)),
        out_shape=jax.ShapeDtypeStruct((N, D), jnp.float32),
    )(xs, wf, dense_b, bias2)


def kernel(nodes, edges, senders, receivers, W, dense_b, bias):
    pad = EP - E
    send3 = jnp.concatenate(
        [senders, jnp.zeros((pad,), senders.dtype)]).reshape(NW, GPW, GL)
    recv3 = jnp.concatenate(
        [receivers, jnp.zeros((pad,), receivers.dtype)]).reshape(NW, GPW, GL)
    wp = jnp.concatenate([edges, jnp.zeros((pad,), edges.dtype)])
    w3 = wp.reshape(NW, GPW, GL)
    w2f = wp.reshape(NW, GPW * GL)

    pdeg = _deg(w3, send3)
    sdeg, scale = _scale_call(pdeg, edges)
    sdeg2 = sdeg.reshape(NP, 1)

    txs = [nodes]
    x = nodes
    prev = None
    for _ in range(1, K):
        p = _matvec(x, w2f, send3, recv3)
        y = _combine(scale, sdeg2, x, p, prev)
        txs.append(y)
        prev, x = x, y

    xs = jnp.stack(txs, axis=1).reshape(N, K * D)
    wf = W.reshape(K * D, D)
    bias2 = bias.reshape(1, D)
    return _matmul(xs, wf, dense_b, bias2)


# speedup vs baseline: 1.0112x; 1.0112x over previous
import jax
import jax.numpy as jnp
from jax import lax
from jax.experimental import pallas as pl
from jax.experimental.pallas import tpu as pltpu
from jax.experimental.pallas import tpu_sc as plsc

NC = 2
NS = 16
NW = NC * NS

N = 10000
E = 320000
D = 128
K = 6

GL = 128
GPW = 80
EP = NW * GPW * GL
NP = 10240
RPT = NP // NS
ZR = 128
CHG = 16
NCH = GPW // CHG

_mesh = plsc.VectorSubcoreMesh(core_axis_name="c", subcore_axis_name="s")


def _matvec_body(x_h, w2f_h, send3_h, recv3_h, p_h,
                 sidx_v, ridx_v, wval0_v, wval1_v, rows_v, acc_sh,
                 gsem, ssem, stsem):
    wvals = (wval0_v, wval1_v)
    c = lax.axis_index("c")
    s = lax.axis_index("s")
    wid = s * NC + c

    def _zb(i, carry):
        for b in range(D // 16):
            rows_v[0, i, pl.ds(b * 16, 16)] = jnp.zeros((16,), jnp.float32)
        return carry
    lax.fori_loop(0, ZR, _zb, 0)
    for i in range(RPT // ZR):
        pltpu.sync_copy(rows_v.at[0], acc_sh.at[pl.ds(s * RPT + i * ZR, ZR)])

    plsc.subcore_barrier()

    def _stage(c):
        slot = c % 2
        return (
            pltpu.async_copy(send3_h.at[wid, pl.ds(c * CHG, CHG)],
                             sidx_v.at[slot], stsem),
            pltpu.async_copy(recv3_h.at[wid, pl.ds(c * CHG, CHG)],
                             ridx_v.at[slot], stsem),
            pltpu.async_copy(w2f_h.at[wid, pl.ds(c * CHG * GL, CHG * GL)],
                             wvals[slot], stsem),
        )

    def _issue_gather(g):
        c, i = g // CHG, g % CHG
        return pltpu.async_copy(
            x_h.at[ridx_v.at[c % 2, i]], rows_v.at[g % 2], gsem)

    def _issue_scatter(g):
        c, i = g // CHG, g % CHG
        return pltpu.async_copy(
            rows_v.at[g % 2], acc_sh.at[sidx_v.at[c % 2, i]], ssem, add=True)

    def _mul_group(g):
        slot, i = (g // CHG) % 2, g % CHG
        p = g % 2

        def _mul(r, c2):
            wb = plsc.load_gather(
                wvals[slot], [jnp.zeros((16,), jnp.int32) + (i * GL + r)])
            for b in range(D // 16):
                rows_v[p, r, pl.ds(b * 16, 16)] = (
                    rows_v[p, r, pl.ds(b * 16, 16)] * wb)
            return c2
        lax.fori_loop(0, GL, _mul, 0)

    for d in _stage(0):
        d.wait()
    gd = _issue_gather(0)
    sd = None
    std = None
    for g in range(GPW):
        gd.wait()
        if sd is not None:
            sd.wait()
        if g % CHG == 1 and g // CHG + 1 < NCH:
            std = _stage(g // CHG + 1)
        if g + 1 < GPW:
            if g % CHG == CHG - 1 and std is not None:
                for d in std:
                    d.wait()
                std = None
            gd = _issue_gather(g + 1)
        _mul_group(g)
        sd = _issue_scatter(g)
    sd.wait()

    plsc.subcore_barrier()
    for i in range(RPT // ZR):
        off = s * RPT + i * ZR
        pltpu.sync_copy(acc_sh.at[pl.ds(off, ZR)], p_h.at[c, pl.ds(off, ZR)])


_matvec = pl.kernel(
    _matvec_body,
    out_type=jax.ShapeDtypeStruct((NC, NP, D), jnp.float32),
    mesh=_mesh,
    compiler_params=pltpu.CompilerParams(needs_layout_passes=False),
    scratch_types=[
        pltpu.VMEM((2, CHG, GL), jnp.int32),
        pltpu.VMEM((2, CHG, GL), jnp.int32),
        pltpu.VMEM((CHG * GL,), jnp.float32),
        pltpu.VMEM((CHG * GL,), jnp.float32),
        pltpu.VMEM((2, GL, D), jnp.float32),
        pltpu.VMEM_SHARED((NP, D), jnp.float32),
        pltpu.SemaphoreType.DMA,
        pltpu.SemaphoreType.DMA,
        pltpu.SemaphoreType.DMA,
    ],
)


def _deg_body(w3_h, send3_h, pdeg_h, sidx_v, wval_v, zv_v, accd_sh):
    c = lax.axis_index("c")
    s = lax.axis_index("s")
    wid = s * NC + c

    def _zb(i, carry):
        zv_v[pl.ds(i * 16, 16)] = jnp.zeros((16,), jnp.float32)
        return carry
    lax.fori_loop(0, RPT // 16, _zb, 0)
    pltpu.sync_copy(zv_v, accd_sh.at[pl.ds(s * RPT, RPT)])

    pltpu.sync_copy(send3_h.at[wid], sidx_v)
    pltpu.sync_copy(w3_h.at[wid], wval_v)
    plsc.subcore_barrier()

    def _grp(j, carry):
        pltpu.sync_copy(wval_v.at[j], accd_sh.at[sidx_v.at[j]], add=True)
        return carry
    lax.fori_loop(0, GPW, _grp, 0)

    plsc.subcore_barrier()
    pltpu.sync_copy(accd_sh.at[pl.ds(s * RPT, RPT)],
                    pdeg_h.at[c, pl.ds(s * RPT, RPT)])


_deg = pl.kernel(
    _deg_body,
    out_type=jax.ShapeDtypeStruct((NC, NP), jnp.float32),
    mesh=_mesh,
    scratch_types=[
        pltpu.VMEM((GPW, GL), jnp.int32),
        pltpu.VMEM((GPW, GL), jnp.float32),
        pltpu.VMEM((RPT,), jnp.float32),
        pltpu.VMEM_SHARED((NP,), jnp.float32),
    ],
)


BM = 1000


def _scale_body(pdeg_ref, edges_ref, sdeg_ref, scale_ref):
    deg = pdeg_ref[0, :] + pdeg_ref[1, :]
    m = jnp.maximum(jnp.max(deg), jnp.max(-edges_ref[...]))
    sc = 1.0 / m
    scale_ref[0, 0] = sc
    sdeg_ref[...] = deg * sc


def _scale_call(pdeg, edges):
    return pl.pallas_call(
        _scale_body,
        out_shape=[
            jax.ShapeDtypeStruct((NP,), jnp.float32),
            jax.ShapeDtypeStruct((1, 1), jnp.float32),
        ],
        out_specs=[
            pl.BlockSpec(memory_space=pltpu.VMEM),
            pl.BlockSpec(memory_space=pltpu.SMEM),
        ],
    )(pdeg, edges)


def _combine1_body(scale_ref, sdeg_ref, x_ref, p_ref, y_ref):
    sc = scale_ref[0, 0]
    ax = p_ref[0] + p_ref[1]
    y_ref[...] = sdeg_ref[...] * x_ref[...] - sc * ax


def _combine2_body(scale_ref, sdeg_ref, x_ref, p_ref, prev_ref, y_ref):
    sc = scale_ref[0, 0]
    ax = p_ref[0] + p_ref[1]
    y_ref[...] = 2.0 * (sdeg_ref[...] * x_ref[...] - sc * ax) - prev_ref[...]


def _combine(scale, sdeg2, x, p, prev=None):
    grid = (N // BM,)
    scale_spec = pl.BlockSpec(memory_space=pltpu.SMEM)
    sdeg_spec = pl.BlockSpec((BM, 1), lambda i: (i, 0))
    row_spec = pl.BlockSpec((BM, D), lambda i: (i, 0))
    p_spec = pl.BlockSpec((NC, BM, D), lambda i: (0, i, 0))
    if prev is None:
        return pl.pallas_call(
            _combine1_body,
            grid=grid,
            in_specs=[scale_spec, sdeg_spec, row_spec, p_spec],
            out_specs=row_spec,
            out_shape=jax.ShapeDtypeStruct((N, D), jnp.float32),
        )(scale, sdeg2, x, p)
    return pl.pallas_call(
        _combine2_body,
        grid=grid,
        in_specs=[scale_spec, sdeg_spec, row_spec, p_spec, row_spec],
        out_specs=row_spec,
        out_shape=jax.ShapeDtypeStruct((N, D), jnp.float32),
    )(scale, sdeg2, x, p, prev)


def _matmul_body(x_ref, w_ref, db_ref, b_ref, o_ref):
    acc = jnp.dot(x_ref[...], w_ref[...], preferred_element_type=jnp.float32)
    o_ref[...] = acc + jnp.sum(db_ref[...], axis=0, keepdims=True) + b_ref[...]


def _matmul(xs, wf, dense_b, bias2):
    grid = (N // BM,)
    return pl.pallas_call(
        _matmul_body,
        grid=grid,
        in_specs=[
            pl.BlockSpec((BM, K * D), lambda i: (i, 0)),
            pl.BlockSpec((K * D, D), lambda i: (0, 0)),
            pl.BlockSpec((K, D), lambda i: (0, 0)),
            pl.BlockSpec((1, D), lambda i: (0, 0)),
        ],
        out_specs=pl.BlockSpec((BM, D), lambda i: (i, 0)),
        out_shape=jax.ShapeDtypeStruct((N, D), jnp.float32),
    )(xs, wf, dense_b, bias2)


def kernel(nodes, edges, senders, receivers, W, dense_b, bias):
    pad = EP - E
    send3 = jnp.concatenate(
        [senders, jnp.zeros((pad,), senders.dtype)]).reshape(NW, GPW, GL)
    recv3 = jnp.concatenate(
        [receivers, jnp.zeros((pad,), receivers.dtype)]).reshape(NW, GPW, GL)
    wp = jnp.concatenate([edges, jnp.zeros((pad,), edges.dtype)])
    w3 = wp.reshape(NW, GPW, GL)
    w2f = wp.reshape(NW, GPW * GL)

    pdeg = _deg(w3, send3)
    sdeg, scale = _scale_call(pdeg, edges)
    sdeg2 = sdeg.reshape(NP, 1)

    txs = [nodes]
    x = nodes
    prev = None
    for _ in range(1, K):
        p = _matvec(x, w2f, send3, recv3)
        y = _combine(scale, sdeg2, x, p, prev)
        txs.append(y)
        prev, x = x, y

    xs = jnp.stack(txs, axis=1).reshape(N, K * D)
    wf = W.reshape(K * D, D)
    bias2 = bias.reshape(1, D)
    return _matmul(xs, wf, dense_b, bias2)


# speedup vs baseline: 1.0259x; 1.0145x over previous
import jax
import jax.numpy as jnp
from jax import lax
from jax.experimental import pallas as pl
from jax.experimental.pallas import tpu as pltpu
from jax.experimental.pallas import tpu_sc as plsc

NC = 2
NS = 16
NW = NC * NS

N = 10000
E = 320000
D = 128
K = 6

GL = 128
GPW = 80
EP = NW * GPW * GL
NP = 10240
RPT = NP // NS
ZR = 128
CHG = 16
NCH = GPW // CHG

_mesh = plsc.VectorSubcoreMesh(core_axis_name="c", subcore_axis_name="s")


def _matvec_body(x_h, w2f_h, send3_h, recv3_h, p_h,
                 sidx_v, ridx_v, wflat_v, rows_v, acc_sh, gsem, ssem, stsem):
    c = lax.axis_index("c")
    s = lax.axis_index("s")
    wid = s * NC + c

    def _zb(i, carry):
        for b in range(D // 16):
            rows_v[0, i, pl.ds(b * 16, 16)] = jnp.zeros((16,), jnp.float32)
        return carry
    lax.fori_loop(0, ZR, _zb, 0)
    for i in range(RPT // ZR):
        pltpu.sync_copy(rows_v.at[0], acc_sh.at[pl.ds(s * RPT + i * ZR, ZR)])

    plsc.subcore_barrier()

    def _stage_issue(ch):
        slot = lax.rem(ch, 2)
        pltpu.async_copy(send3_h.at[wid, pl.ds(ch * CHG, CHG)],
                         sidx_v.at[slot], stsem)
        pltpu.async_copy(recv3_h.at[wid, pl.ds(ch * CHG, CHG)],
                         ridx_v.at[slot], stsem)
        pltpu.async_copy(w2f_h.at[wid, pl.ds(ch * CHG * GL, CHG * GL)],
                         wflat_v.at[pl.ds(slot * CHG * GL, CHG * GL)], stsem)

    def _stage_wait():
        for _ in range(3):
            pltpu.make_async_copy(send3_h.at[wid, pl.ds(0, CHG)],
                                  sidx_v.at[0], stsem).wait()

    def _gissue(g, p):
        slot = lax.rem(g // CHG, 2)
        i = lax.rem(g, CHG)
        pltpu.async_copy(x_h.at[ridx_v.at[slot, i]], rows_v.at[p], gsem)

    def _gwait(p):
        pltpu.make_async_copy(x_h.at[ridx_v.at[0, 0]],
                              rows_v.at[p], gsem).wait()

    def _sissue(g, p):
        slot = lax.rem(g // CHG, 2)
        i = lax.rem(g, CHG)
        pltpu.async_copy(rows_v.at[p], acc_sh.at[sidx_v.at[slot, i]],
                         ssem, add=True)

    def _swait(p):
        pltpu.make_async_copy(rows_v.at[p],
                              acc_sh.at[sidx_v.at[0, 0]], ssem).wait()

    def _mul(g, p):
        slot = lax.rem(g // CHG, 2)
        i = lax.rem(g, CHG)
        base = slot * (CHG * GL) + i * GL

        def _r(r, cy):
            wb = plsc.load_gather(
                wflat_v, [jnp.zeros((16,), jnp.int32) + (base + r)])
            for b in range(D // 16):
                rows_v[p, r, pl.ds(b * 16, 16)] = (
                    rows_v[p, r, pl.ds(b * 16, 16)] * wb)
            return cy
        lax.fori_loop(0, GL, _r, 0)

    _stage_issue(0)
    _stage_wait()
    _gissue(0, 0)

    def _pair(t, carry):
        g0 = 2 * t
        g1 = g0 + 1
        _gwait(0)

        @pl.when(t > 0)
        def _():
            _swait(1)

        @pl.when(jnp.logical_and(lax.rem(t, 8) == 0, t < (NCH - 1) * 8))
        def _():
            _stage_issue(t // 8 + 1)

        _gissue(g1, 1)
        _mul(g0, 0)
        _sissue(g0, 0)

        _gwait(1)
        _swait(0)

        @pl.when(jnp.logical_and(lax.rem(t, 8) == 7, g1 + 1 < GPW))
        def _():
            _stage_wait()

        @pl.when(g1 + 1 < GPW)
        def _():
            _gissue(g1 + 1, 0)

        _mul(g1, 1)
        _sissue(g1, 1)
        return carry
    lax.fori_loop(0, GPW // 2, _pair, 0)
    _swait(1)

    plsc.subcore_barrier()
    for i in range(RPT // ZR):
        off = s * RPT + i * ZR
        pltpu.sync_copy(acc_sh.at[pl.ds(off, ZR)], p_h.at[c, pl.ds(off, ZR)])


_matvec = pl.kernel(
    _matvec_body,
    out_type=jax.ShapeDtypeStruct((NC, NP, D), jnp.float32),
    mesh=_mesh,
    compiler_params=pltpu.CompilerParams(needs_layout_passes=False),
    scratch_types=[
        pltpu.VMEM((2, CHG, GL), jnp.int32),
        pltpu.VMEM((2, CHG, GL), jnp.int32),
        pltpu.VMEM((2 * CHG * GL,), jnp.float32),
        pltpu.VMEM((2, GL, D), jnp.float32),
        pltpu.VMEM_SHARED((NP, D), jnp.float32),
        pltpu.SemaphoreType.DMA,
        pltpu.SemaphoreType.DMA,
        pltpu.SemaphoreType.DMA,
    ],
)


def _deg_body(w3_h, send3_h, pdeg_h, sidx_v, wval_v, zv_v, accd_sh):
    c = lax.axis_index("c")
    s = lax.axis_index("s")
    wid = s * NC + c

    def _zb(i, carry):
        zv_v[pl.ds(i * 16, 16)] = jnp.zeros((16,), jnp.float32)
        return carry
    lax.fori_loop(0, RPT // 16, _zb, 0)
    pltpu.sync_copy(zv_v, accd_sh.at[pl.ds(s * RPT, RPT)])

    pltpu.sync_copy(send3_h.at[wid], sidx_v)
    pltpu.sync_copy(w3_h.at[wid], wval_v)
    plsc.subcore_barrier()

    def _grp(j, carry):
        pltpu.sync_copy(wval_v.at[j], accd_sh.at[sidx_v.at[j]], add=True)
        return carry
    lax.fori_loop(0, GPW, _grp, 0)

    plsc.subcore_barrier()
    pltpu.sync_copy(accd_sh.at[pl.ds(s * RPT, RPT)],
                    pdeg_h.at[c, pl.ds(s * RPT, RPT)])


_deg = pl.kernel(
    _deg_body,
    out_type=jax.ShapeDtypeStruct((NC, NP), jnp.float32),
    mesh=_mesh,
    scratch_types=[
        pltpu.VMEM((GPW, GL), jnp.int32),
        pltpu.VMEM((GPW, GL), jnp.float32),
        pltpu.VMEM((RPT,), jnp.float32),
        pltpu.VMEM_SHARED((NP,), jnp.float32),
    ],
)


BM = 1000


def _scale_body(pdeg_ref, edges_ref, sdeg_ref, scale_ref):
    deg = pdeg_ref[0, :] + pdeg_ref[1, :]
    m = jnp.maximum(jnp.max(deg), jnp.max(-edges_ref[...]))
    sc = 1.0 / m
    scale_ref[0, 0] = sc
    sdeg_ref[...] = deg * sc


def _scale_call(pdeg, edges):
    return pl.pallas_call(
        _scale_body,
        out_shape=[
            jax.ShapeDtypeStruct((NP,), jnp.float32),
            jax.ShapeDtypeStruct((1, 1), jnp.float32),
        ],
        out_specs=[
            pl.BlockSpec(memory_space=pltpu.VMEM),
            pl.BlockSpec(memory_space=pltpu.SMEM),
        ],
    )(pdeg, edges)


def _combine1_body(scale_ref, sdeg_ref, x_ref, p_ref, y_ref):
    sc = scale_ref[0, 0]
    ax = p_ref[0] + p_ref[1]
    y_ref[...] = sdeg_ref[...] * x_ref[...] - sc * ax


def _combine2_body(scale_ref, sdeg_ref, x_ref, p_ref, prev_ref, y_ref):
    sc = scale_ref[0, 0]
    ax = p_ref[0] + p_ref[1]
    y_ref[...] = 2.0 * (sdeg_ref[...] * x_ref[...] - sc * ax) - prev_ref[...]


def _combine(scale, sdeg2, x, p, prev=None):
    grid = (N // BM,)
    scale_spec = pl.BlockSpec(memory_space=pltpu.SMEM)
    sdeg_spec = pl.BlockSpec((BM, 1), lambda i: (i, 0))
    row_spec = pl.BlockSpec((BM, D), lambda i: (i, 0))
    p_spec = pl.BlockSpec((NC, BM, D), lambda i: (0, i, 0))
    if prev is None:
        return pl.pallas_call(
            _combine1_body,
            grid=grid,
            in_specs=[scale_spec, sdeg_spec, row_spec, p_spec],
            out_specs=row_spec,
            out_shape=jax.ShapeDtypeStruct((N, D), jnp.float32),
        )(scale, sdeg2, x, p)
    return pl.pallas_call(
        _combine2_body,
        grid=grid,
        in_specs=[scale_spec, sdeg_spec, row_spec, p_spec, row_spec],
        out_specs=row_spec,
        out_shape=jax.ShapeDtypeStruct((N, D), jnp.float32),
    )(scale, sdeg2, x, p, prev)


def _matmul_body(x_ref, w_ref, db_ref, b_ref, o_ref):
    acc = jnp.dot(x_ref[...], w_ref[...], preferred_element_type=jnp.float32)
    o_ref[...] = acc + jnp.sum(db_ref[...], axis=0, keepdims=True) + b_ref[...]


def _matmul(xs, wf, dense_b, bias2):
    grid = (N // BM,)
    return pl.pallas_call(
        _matmul_body,
        grid=grid,
        in_specs=[
            pl.BlockSpec((BM, K * D), lambda i: (i, 0)),
            pl.BlockSpec((K * D, D), lambda i: (0, 0)),
            pl.BlockSpec((K, D), lambda i: (0, 0)),
            pl.BlockSpec((1, D), lambda i: (0, 0)),
        ],
        out_specs=pl.BlockSpec((BM, D), lambda i: (i, 0)),
        out_shape=jax.ShapeDtypeStruct((N, D), jnp.float32),
    )(xs, wf, dense_b, bias2)


def kernel(nodes, edges, senders, receivers, W, dense_b, bias):
    pad = EP - E
    send3 = jnp.concatenate(
        [senders, jnp.zeros((pad,), senders.dtype)]).reshape(NW, GPW, GL)
    recv3 = jnp.concatenate(
        [receivers, jnp.zeros((pad,), receivers.dtype)]).reshape(NW, GPW, GL)
    wp = jnp.concatenate([edges, jnp.zeros((pad,), edges.dtype)])
    w3 = wp.reshape(NW, GPW, GL)
    w2f = wp.reshape(NW, GPW * GL)

    pdeg = _deg(w3, send3)
    sdeg, scale = _scale_call(pdeg, edges)
    sdeg2 = sdeg.reshape(NP, 1)

    txs = [nodes]
    x = nodes
    prev = None
    for _ in range(1, K):
        p = _matvec(x, w2f, send3, recv3)
        y = _combine(scale, sdeg2, x, p, prev)
        txs.append(y)
        prev, x = x, y

    xs = jnp.stack(txs, axis=1).reshape(N, K * D)
    wf = W.reshape(K * D, D)
    bias2 = bias.reshape(1, D)
    return _matmul(xs, wf, dense_b, bias2)
